# Initial kernel scaffold; baseline (speedup 1.0000x reference)
import jax
import jax.numpy as jnp
from jax.experimental import pallas as pl


def kernel(x, edge_index, beta):
    raise NotImplementedError("write your pallas kernel here")


# speedup vs baseline: 6.1560x; 6.1560x over previous
import functools

import jax
import jax.numpy as jnp
from jax import lax
from jax.experimental import pallas as pl
from jax.experimental.pallas import tpu as pltpu
from jax.experimental.pallas import tpu_sc as plsc

L = 16
NC = 2
NS = 16
NW = NC * NS
C = 128
GPC = C // L


def _rotg(v, idx):
    return lax.gather(
        v, idx[:, None],
        dimension_numbers=lax.GatherDimensionNumbers(
            offset_dims=(), collapsed_slice_dims=(0,), start_index_map=(0,)),
        slice_sizes=(1,),
        mode=lax.GatherScatterMode.PROMISE_IN_BOUNDS)


def _norm_body(x_ref, o_ref):
    x = x_ref[...]
    ss = jnp.sum(x * x, axis=1, keepdims=True)
    nrm = jnp.maximum(jnp.sqrt(ss), 1e-12)
    o_ref[...] = x / nrm


def _final_body(x_ref, a0_ref, a1_ref, d0_ref, d1_ref, o_ref):
    num = a0_ref[...] + a1_ref[...]
    den = jnp.maximum(d0_ref[...] + d1_ref[...], 1e-16)
    o_ref[...] = x_ref[...] + jnp.maximum(num / den, 0.0)


def _phase1_body(cpt, d_dim, xn, srcf, dstf, beta16_hbm, den_out, ex_out,
                 idxs_v, idxd_v, rows_s, rows_d, ex_v, exrow_v, beta_v,
                 den_sh, sem_s, sem_d):
    c = lax.axis_index("c")
    s = lax.axis_index("s")
    w = s * NC + c
    npad = den_sh.shape[0]
    per_tile = npad // NS
    iota = lax.broadcasted_iota(jnp.int32, (L,), 0)

    def zfill(i, carry):
        for k in range(d_dim // L):
            exrow_v[i, pl.ds(k * L, L)] = jnp.zeros((L,), jnp.float32)
        return carry
    lax.fori_loop(0, C, zfill, 0)

    for off in range(0, per_tile, C):
        wdt = min(C, per_tile - off)
        pltpu.sync_copy(exrow_v.at[pl.ds(0, wdt)],
                        den_sh.at[pl.ds(s * per_tile + off, wdt)])
    plsc.subcore_barrier()

    pltpu.sync_copy(beta16_hbm, beta_v)
    beta16 = beta_v[...]
    tile_base = w * (cpt * C)

    def chunk_body(ch, carry):
        base = tile_base + ch * C
        pltpu.sync_copy(srcf.at[pl.ds(base, C)], idxs_v)
        pltpu.sync_copy(dstf.at[pl.ds(base, C)], idxd_v)
        cp_s = pltpu.async_copy(xn.at[idxs_v], rows_s, sem_s)
        cp_d = pltpu.async_copy(xn.at[idxd_v], rows_d, sem_d)
        cp_s.wait()
        cp_d.wait()

        def group(g, carry2):
            def edge(j, exv):
                e = g * L + j
                acc = rows_s[e, pl.ds(0, L)] * rows_d[e, pl.ds(0, L)]
                for k in range(1, d_dim // L):
                    acc = acc + (rows_s[e, pl.ds(k * L, L)] *
                                 rows_d[e, pl.ds(k * L, L)])
                for r in (8, 4, 2, 1):
                    acc = acc + _rotg(acc, (iota + r) & (L - 1))
                spl = jnp.exp(acc * beta16)
                for k in range(d_dim // L):
                    exrow_v[e, pl.ds(k * L, L)] = spl
                return jnp.where(iota == j, spl, exv)
            exv = lax.fori_loop(0, L, edge, jnp.zeros((L,), jnp.float32))
            ex_v[pl.ds(g * L, L)] = exv
            return carry2
        lax.fori_loop(0, GPC, group, 0)

        pltpu.sync_copy(ex_v, ex_out.at[pl.ds(base, C)])
        pltpu.sync_copy(exrow_v, den_sh.at[idxd_v], add=True)
        return carry

    lax.fori_loop(0, cpt, chunk_body, 0)

    plsc.subcore_barrier()
    pltpu.sync_copy(den_sh.at[pl.ds(s * per_tile, per_tile)],
                    den_out.at[c, pl.ds(s * per_tile, per_tile)])


def _phase2_body(cpt, d_dim, xpad, srcf, dstf, ex_hbm, agg_out,
                 idxs_v, idxd_v, rows_v, ex_v,
                 agg_sh, sem_r):
    c = lax.axis_index("c")
    s = lax.axis_index("s")
    w = s * NC + c
    npad = agg_sh.shape[0]
    per_tile = npad // NS

    def zfill(i, carry):
        for k in range(d_dim // L):
            rows_v[i, pl.ds(k * L, L)] = jnp.zeros((L,), jnp.float32)
        return carry
    lax.fori_loop(0, C, zfill, 0)

    for off in range(0, per_tile, C):
        wdt = min(C, per_tile - off)
        pltpu.sync_copy(rows_v.at[pl.ds(0, wdt)],
                        agg_sh.at[pl.ds(s * per_tile + off, wdt)])
    plsc.subcore_barrier()

    tile_base = w * (cpt * C)

    def chunk_body(ch, carry):
        base = tile_base + ch * C
        pltpu.sync_copy(srcf.at[pl.ds(base, C)], idxs_v)
        pltpu.sync_copy(dstf.at[pl.ds(base, C)], idxd_v)
        cp_r = pltpu.async_copy(xpad.at[idxs_v], rows_v, sem_r)
        pltpu.sync_copy(ex_hbm.at[pl.ds(base, C)], ex_v)
        cp_r.wait()

        def group(g, carry2):
            exg = ex_v[pl.ds(g * L, L)]

            def edge(j, carry3):
                e = g * L + j
                spl = _rotg(exg, jnp.full((L,), j, jnp.int32))
                for k in range(d_dim // L):
                    v = rows_v[e, pl.ds(k * L, L)]
                    rows_v[e, pl.ds(k * L, L)] = v * spl
                return carry3
            lax.fori_loop(0, L, edge, 0)
            return carry2
        lax.fori_loop(0, GPC, group, 0)

        pltpu.sync_copy(rows_v, agg_sh.at[idxd_v], add=True)
        return carry

    lax.fori_loop(0, cpt, chunk_body, 0)

    plsc.subcore_barrier()
    pltpu.sync_copy(agg_sh.at[pl.ds(s * per_tile, per_tile)],
                    agg_out.at[c, pl.ds(s * per_tile, per_tile)])


def kernel(x, edge_index, beta):
    n, d = x.shape
    e = edge_index.shape[1]
    et = e + n
    cpt = -(-et // (NW * C))
    etpad = NW * C * cpt
    pad_e = etpad - et
    npad = (NS * 8) * (-(-(n + 1) // (NS * 8)))

    src = edge_index[0].astype(jnp.int32)
    dst = edge_index[1].astype(jnp.int32)
    loops = jnp.arange(n, dtype=jnp.int32)
    srcf = jnp.concatenate([src, loops, jnp.zeros((pad_e,), jnp.int32)])
    dstf = jnp.concatenate([dst, loops, jnp.full((pad_e,), n, jnp.int32)])
    xpad = jnp.pad(x, ((0, npad - n), (0, 0)))
    beta16 = jnp.broadcast_to(beta.astype(jnp.float32), (L,))

    rows_blk = 128
    xn = pl.pallas_call(
        _norm_body,
        grid=(npad // rows_blk,),
        in_specs=[pl.BlockSpec((rows_blk, d), lambda i: (i, 0))],
        out_specs=pl.BlockSpec((rows_blk, d), lambda i: (i, 0)),
        out_shape=jax.ShapeDtypeStruct((npad, d), jnp.float32),
    )(xpad)

    mesh = plsc.VectorSubcoreMesh(
        core_axis_name="c", subcore_axis_name="s",
        num_cores=NC, num_subcores=NS)

    phase1 = pl.kernel(
        functools.partial(_phase1_body, cpt, d),
        out_type=(jax.ShapeDtypeStruct((NC, npad, d), jnp.float32),
                  jax.ShapeDtypeStruct((etpad,), jnp.float32)),
        mesh=mesh,
        scratch_types=[
            pltpu.VMEM((C,), jnp.int32),
            pltpu.VMEM((C,), jnp.int32),
            pltpu.VMEM((C, d), jnp.float32),
            pltpu.VMEM((C, d), jnp.float32),
            pltpu.VMEM((C,), jnp.float32),
            pltpu.VMEM((C, d), jnp.float32),
            pltpu.VMEM((L,), jnp.float32),
            pltpu.VMEM_SHARED((npad, d), jnp.float32),
            pltpu.SemaphoreType.DMA,
            pltpu.SemaphoreType.DMA,
        ],
    )
    denp, ex = phase1(xn, srcf, dstf, beta16)

    phase2 = pl.kernel(
        functools.partial(_phase2_body, cpt, d),
        out_type=jax.ShapeDtypeStruct((NC, npad, d), jnp.float32),
        mesh=mesh,
        scratch_types=[
            pltpu.VMEM((C,), jnp.int32),
            pltpu.VMEM((C,), jnp.int32),
            pltpu.VMEM((C, d), jnp.float32),
            pltpu.VMEM((C,), jnp.float32),
            pltpu.VMEM_SHARED((npad, d), jnp.float32),
            pltpu.SemaphoreType.DMA,
        ],
    )
    aggp = phase2(xpad, srcf, dstf, ex)

    fb = 80
    out = pl.pallas_call(
        _final_body,
        grid=(n // fb,),
        in_specs=[pl.BlockSpec((fb, d), lambda i: (i, 0))] * 5,
        out_specs=pl.BlockSpec((fb, d), lambda i: (i, 0)),
        out_shape=jax.ShapeDtypeStruct((n, d), jnp.float32),
    )(x, aggp[0, :n], aggp[1, :n], denp[0, :n], denp[1, :n])
    return out


# speedup vs baseline: 6.6702x; 1.0835x over previous
import functools

import jax
import jax.numpy as jnp
from jax import lax
from jax.experimental import pallas as pl
from jax.experimental.pallas import tpu as pltpu
from jax.experimental.pallas import tpu_sc as plsc

L = 16
NC = 2
NS = 16
NW = NC * NS
C = 128
GPC = C // L


def _rotg(v, idx):
    return lax.gather(
        v, idx[:, None],
        dimension_numbers=lax.GatherDimensionNumbers(
            offset_dims=(), collapsed_slice_dims=(0,), start_index_map=(0,)),
        slice_sizes=(1,),
        mode=lax.GatherScatterMode.PROMISE_IN_BOUNDS)


def _norm_body(x_ref, o_ref):
    x = x_ref[...]
    ss = jnp.sum(x * x, axis=1, keepdims=True)
    nrm = jnp.maximum(jnp.sqrt(ss), 1e-12)
    o_ref[...] = x / nrm


def _final_body(x_ref, a0_ref, a1_ref, d0_ref, d1_ref, o_ref):
    num = a0_ref[...] + a1_ref[...]
    den = jnp.maximum(d0_ref[...] + d1_ref[...], 1e-16)
    o_ref[...] = x_ref[...] + jnp.maximum(num / den, 0.0)


def _phase1_body(cpt, d_dim, xn, srcf, dstf, beta16_hbm, den_out, ex_out,
                 idxs0, idxd0, idxs1, idxd1, idxq0, idxq1, rs0, rd0, rs1, rd1,
                 ex_v, exrow_v, beta_v,
                 den_sh, sem_s0, sem_d0, sem_s1, sem_d1):
    c = lax.axis_index("c")
    s = lax.axis_index("s")
    w = s * NC + c
    nq = den_sh.shape[0]
    per_tile = nq // NS
    iota = lax.broadcasted_iota(jnp.int32, (L,), 0)

    def zfill(i, carry):
        for k in range(d_dim // L):
            exrow_v[i, pl.ds(k * L, L)] = jnp.zeros((L,), jnp.float32)
        return carry
    lax.fori_loop(0, C, zfill, 0)

    for off in range(0, per_tile, C):
        wdt = min(C, per_tile - off)
        pltpu.sync_copy(exrow_v.at[pl.ds(0, wdt)],
                        den_sh.at[pl.ds(s * per_tile + off, wdt)])
    plsc.subcore_barrier()

    pltpu.sync_copy(beta16_hbm, beta_v)
    beta16 = beta_v[...]
    tile_base = w * (cpt * C)

    def stage(ch, idxs, idxd, rs, rd, sem_s, sem_d):
        base = tile_base + ch * C
        pltpu.sync_copy(srcf.at[pl.ds(base, C)], idxs)
        pltpu.sync_copy(dstf.at[pl.ds(base, C)], idxd)
        pltpu.async_copy(xn.at[idxs], rs, sem_s)
        pltpu.async_copy(xn.at[idxd], rd, sem_d)

    def wait(idxs, idxd, rs, rd, sem_s, sem_d):
        pltpu.make_async_copy(xn.at[idxs], rs, sem_s).wait()
        pltpu.make_async_copy(xn.at[idxd], rd, sem_d).wait()

    def compute(ch, rows_s, rows_d, idxd, idxq):
        def group(g, carry2):
            dst16 = idxd[pl.ds(g * L, L)]
            idxq[pl.ds(g * L, L)] = lax.shift_right_logical(dst16, 3)
            exv = jnp.zeros((L,), jnp.float32)
            for j in range(L):
                e_ref = g * L + j
                acc = rows_s[e_ref, pl.ds(0, L)] * rows_d[e_ref, pl.ds(0, L)]
                for k in range(1, d_dim // L):
                    acc = acc + (rows_s[e_ref, pl.ds(k * L, L)] *
                                 rows_d[e_ref, pl.ds(k * L, L)])
                for r in (8, 4, 2, 1):
                    acc = acc + _rotg(acc, (iota + r) & (L - 1))
                spl = jnp.exp(acc * beta16)
                blk = _rotg(dst16, jnp.full((L,), j, jnp.int32)) & 7
                blkf = blk.astype(jnp.float32)
                for k in range(d_dim // L):
                    mk = jnp.maximum(
                        1.0 - jnp.abs(blkf - jnp.float32(k)), 0.0)
                    exrow_v[e_ref, pl.ds(k * L, L)] = spl * mk
                exv = jnp.where(iota == j, spl, exv)
            ex_v[pl.ds(g * L, L)] = exv
            return carry2
        lax.fori_loop(0, GPC, group, 0)
        pltpu.sync_copy(ex_v, ex_out.at[pl.ds(tile_base + ch * C, C)])
        pltpu.sync_copy(exrow_v, den_sh.at[idxq], add=True)

    half = cpt // 2
    stage(0, idxs0, idxd0, rs0, rd0, sem_s0, sem_d0)

    def pair(t, carry):
        ch0 = 2 * t
        wait(idxs0, idxd0, rs0, rd0, sem_s0, sem_d0)
        stage(ch0 + 1, idxs1, idxd1, rs1, rd1, sem_s1, sem_d1)
        compute(ch0, rs0, rd0, idxd0, idxq0)
        wait(idxs1, idxd1, rs1, rd1, sem_s1, sem_d1)

        @pl.when(t + 1 < half)
        def _():
            stage(ch0 + 2, idxs0, idxd0, rs0, rd0, sem_s0, sem_d0)
        compute(ch0 + 1, rs1, rd1, idxd1, idxq1)
        return carry

    lax.fori_loop(0, half, pair, 0)

    plsc.subcore_barrier()
    pltpu.sync_copy(den_sh.at[pl.ds(s * per_tile, per_tile)],
                    den_out.at[c, pl.ds(s * per_tile, per_tile)])


def _phase2_body(cpt, d_dim, xpad, srcf, dstf, ex_hbm, agg_out,
                 idxs0, idxd0, idxs1, idxd1, r0, r1, ex_v,
                 agg_sh, sem_r0, sem_r1):
    c = lax.axis_index("c")
    s = lax.axis_index("s")
    w = s * NC + c
    npad = agg_sh.shape[0]
    per_tile = npad // NS

    def zfill(i, carry):
        for k in range(d_dim // L):
            r0[i, pl.ds(k * L, L)] = jnp.zeros((L,), jnp.float32)
        return carry
    lax.fori_loop(0, C, zfill, 0)

    for off in range(0, per_tile, C):
        wdt = min(C, per_tile - off)
        pltpu.sync_copy(r0.at[pl.ds(0, wdt)],
                        agg_sh.at[pl.ds(s * per_tile + off, wdt)])
    plsc.subcore_barrier()

    tile_base = w * (cpt * C)

    def stage(ch, idxs, idxd, rr, sem_r):
        base = tile_base + ch * C
        pltpu.sync_copy(srcf.at[pl.ds(base, C)], idxs)
        pltpu.sync_copy(dstf.at[pl.ds(base, C)], idxd)
        pltpu.async_copy(xpad.at[idxs], rr, sem_r)

    def compute(ch, rows_v, idxd):
        pltpu.sync_copy(ex_hbm.at[pl.ds(tile_base + ch * C, C)], ex_v)

        def group(g, carry2):
            exg = ex_v[pl.ds(g * L, L)]
            for j in range(L):
                e_ref = g * L + j
                spl = _rotg(exg, jnp.full((L,), j, jnp.int32))
                for k in range(d_dim // L):
                    v = rows_v[e_ref, pl.ds(k * L, L)]
                    rows_v[e_ref, pl.ds(k * L, L)] = v * spl
            return carry2
        lax.fori_loop(0, GPC, group, 0)
        pltpu.sync_copy(rows_v, agg_sh.at[idxd], add=True)

    half = cpt // 2
    stage(0, idxs0, idxd0, r0, sem_r0)

    def pair(t, carry):
        ch0 = 2 * t
        pltpu.make_async_copy(xpad.at[idxs0], r0, sem_r0).wait()
        stage(ch0 + 1, idxs1, idxd1, r1, sem_r1)
        compute(ch0, r0, idxd0)
        pltpu.make_async_copy(xpad.at[idxs1], r1, sem_r1).wait()

        @pl.when(t + 1 < half)
        def _():
            stage(ch0 + 2, idxs0, idxd0, r0, sem_r0)
        compute(ch0 + 1, r1, idxd1)
        return carry

    lax.fori_loop(0, half, pair, 0)

    plsc.subcore_barrier()
    pltpu.sync_copy(agg_sh.at[pl.ds(s * per_tile, per_tile)],
                    agg_out.at[c, pl.ds(s * per_tile, per_tile)])


def kernel(x, edge_index, beta):
    n, d = x.shape
    e = edge_index.shape[1]
    et = e + n
    cpt = -(-et // (NW * C))
    cpt = cpt + (cpt & 1)
    etpad = NW * C * cpt
    pad_e = etpad - et
    npad = (NS * 8) * (-(-(n + 1) // (NS * 8)))

    src = edge_index[0].astype(jnp.int32)
    dst = edge_index[1].astype(jnp.int32)
    loops = jnp.arange(n, dtype=jnp.int32)
    srcf = jnp.concatenate([src, loops, jnp.zeros((pad_e,), jnp.int32)])
    dstf = jnp.concatenate([dst, loops, jnp.full((pad_e,), n, jnp.int32)])
    xpad = jnp.pad(x, ((0, npad - n), (0, 0)))
    beta16 = jnp.broadcast_to(beta.astype(jnp.float32), (L,))

    rows_blk = 128
    xn = pl.pallas_call(
        _norm_body,
        grid=(npad // rows_blk,),
        in_specs=[pl.BlockSpec((rows_blk, d), lambda i: (i, 0))],
        out_specs=pl.BlockSpec((rows_blk, d), lambda i: (i, 0)),
        out_shape=jax.ShapeDtypeStruct((npad, d), jnp.float32),
    )(xpad)

    mesh = plsc.VectorSubcoreMesh(
        core_axis_name="c", subcore_axis_name="s",
        num_cores=NC, num_subcores=NS)

    nq = (NS * 8) * (-(-(npad // 8) // (NS * 8)))

    phase1 = pl.kernel(
        functools.partial(_phase1_body, cpt, d),
        out_type=(jax.ShapeDtypeStruct((NC, nq, d), jnp.float32),
                  jax.ShapeDtypeStruct((etpad,), jnp.float32)),
        mesh=mesh,
        scratch_types=[
            pltpu.VMEM((C,), jnp.int32),
            pltpu.VMEM((C,), jnp.int32),
            pltpu.VMEM((C,), jnp.int32),
            pltpu.VMEM((C,), jnp.int32),
            pltpu.VMEM((C,), jnp.int32),
            pltpu.VMEM((C,), jnp.int32),
            pltpu.VMEM((C, d), jnp.float32),
            pltpu.VMEM((C, d), jnp.float32),
            pltpu.VMEM((C, d), jnp.float32),
            pltpu.VMEM((C, d), jnp.float32),
            pltpu.VMEM((C,), jnp.float32),
            pltpu.VMEM((C, d), jnp.float32),
            pltpu.VMEM((L,), jnp.float32),
            pltpu.VMEM_SHARED((nq, d), jnp.float32),
            pltpu.SemaphoreType.DMA,
            pltpu.SemaphoreType.DMA,
            pltpu.SemaphoreType.DMA,
            pltpu.SemaphoreType.DMA,
        ],
    )
    denp, ex = phase1(xn, srcf, dstf, beta16)
    dencol = jnp.broadcast_to(
        denp.reshape(NC, nq * d)[:, ::L][:, :n, None], (NC, n, d))

    phase2 = pl.kernel(
        functools.partial(_phase2_body, cpt, d),
        out_type=jax.ShapeDtypeStruct((NC, npad, d), jnp.float32),
        mesh=mesh,
        scratch_types=[
            pltpu.VMEM((C,), jnp.int32),
            pltpu.VMEM((C,), jnp.int32),
            pltpu.VMEM((C,), jnp.int32),
            pltpu.VMEM((C,), jnp.int32),
            pltpu.VMEM((C, d), jnp.float32),
            pltpu.VMEM((C, d), jnp.float32),
            pltpu.VMEM((C,), jnp.float32),
            pltpu.VMEM_SHARED((npad, d), jnp.float32),
            pltpu.SemaphoreType.DMA,
            pltpu.SemaphoreType.DMA,
        ],
    )
    aggp = phase2(xpad, srcf, dstf, ex)

    fb = 80
    out = pl.pallas_call(
        _final_body,
        grid=(n // fb,),
        in_specs=[pl.BlockSpec((fb, d), lambda i: (i, 0))] * 5,
        out_specs=pl.BlockSpec((fb, d), lambda i: (i, 0)),
        out_shape=jax.ShapeDtypeStruct((n, d), jnp.float32),
    )(x, aggp[0, :n], aggp[1, :n], dencol[0], dencol[1])
    return out


# speedup vs baseline: 7.2485x; 1.0867x over previous
import functools

import jax
import jax.numpy as jnp
from jax import lax
from jax.experimental import pallas as pl
from jax.experimental.pallas import tpu as pltpu
from jax.experimental.pallas import tpu_sc as plsc

L = 16
NC = 2
NS = 16
NW = NC * NS
C = 128
GPC = C // L


def _rotg(v, idx):
    return lax.gather(
        v, idx[:, None],
        dimension_numbers=lax.GatherDimensionNumbers(
            offset_dims=(), collapsed_slice_dims=(0,), start_index_map=(0,)),
        slice_sizes=(1,),
        mode=lax.GatherScatterMode.PROMISE_IN_BOUNDS)


def _norm_body(x_ref, o_ref):
    x = x_ref[...]
    ss = jnp.sum(x * x, axis=1, keepdims=True)
    nrm = jnp.maximum(jnp.sqrt(ss), 1e-12)
    o_ref[...] = x / nrm


def _final_body(x_ref, a0_ref, a1_ref, d0_ref, d1_ref, o_ref):
    num = a0_ref[...] + a1_ref[...]
    den = jnp.maximum(d0_ref[...] + d1_ref[...], 1e-16)
    o_ref[...] = x_ref[...] + jnp.maximum(num / den, 0.0)


def _phase1_body(cpt, d_dim, xn, srcf, dstf, beta16_hbm, den_out, ex_out,
                 src_all, idxd0, idxd1, idxq0, idxq1, rs0, rd0, rs1, rd1,
                 exv0, exv1, exrow0, exrow1, beta_v,
                 den_sh, sem_s0, sem_d0, sem_s1, sem_d1, sem_w0, sem_w1,
                 sem_e0, sem_e1):
    c = lax.axis_index("c")
    s = lax.axis_index("s")
    w = s * NC + c
    nq = den_sh.shape[0]
    per_tile = nq // NS
    iota = lax.broadcasted_iota(jnp.int32, (L,), 0)

    def zfill(i, carry):
        for k in range(d_dim // L):
            exrow0[i, pl.ds(k * L, L)] = jnp.zeros((L,), jnp.float32)
        return carry
    lax.fori_loop(0, C, zfill, 0)

    for off in range(0, per_tile, C):
        wdt = min(C, per_tile - off)
        pltpu.sync_copy(exrow0.at[pl.ds(0, wdt)],
                        den_sh.at[pl.ds(s * per_tile + off, wdt)])
    plsc.subcore_barrier()

    pltpu.sync_copy(beta16_hbm, beta_v)
    beta16 = beta_v[...]
    tile_base = w * (cpt * C)
    pltpu.sync_copy(srcf.at[pl.ds(tile_base, cpt * C)], src_all)

    def gathers(ch, idxd, rs, rd, sem_s, sem_d):
        pltpu.sync_copy(dstf.at[pl.ds(tile_base + ch * C, C)], idxd)
        pltpu.async_copy(xn.at[src_all.at[pl.ds(ch * C, C)]], rs, sem_s)
        pltpu.async_copy(xn.at[idxd], rd, sem_d)

    def gwait(ch, idxd, rs, rd, sem_s, sem_d):
        pltpu.make_async_copy(xn.at[src_all.at[pl.ds(ch * C, C)]],
                              rs, sem_s).wait()
        pltpu.make_async_copy(xn.at[idxd], rd, sem_d).wait()

    def compute(ch, rows_s, rows_d, idxd, idxq, ex_v, exrow_v, sem_w, sem_e):
        def group(g, carry2):
            dst16 = idxd[pl.ds(g * L, L)]
            idxq[pl.ds(g * L, L)] = lax.shift_right_logical(dst16, 3)
            exv = jnp.zeros((L,), jnp.float32)
            for j in range(L):
                e_ref = g * L + j
                acc = rows_s[e_ref, pl.ds(0, L)] * rows_d[e_ref, pl.ds(0, L)]
                for k in range(1, d_dim // L):
                    acc = acc + (rows_s[e_ref, pl.ds(k * L, L)] *
                                 rows_d[e_ref, pl.ds(k * L, L)])
                for r in (8, 4, 2, 1):
                    acc = acc + _rotg(acc, (iota + r) & (L - 1))
                spl = jnp.exp(acc * beta16)
                blk = _rotg(dst16, jnp.full((L,), j, jnp.int32)) & 7
                blkf = blk.astype(jnp.float32)
                for k in range(d_dim // L):
                    mk = jnp.maximum(
                        1.0 - jnp.abs(blkf - jnp.float32(k)), 0.0)
                    exrow_v[e_ref, pl.ds(k * L, L)] = spl * mk
                exv = jnp.where(iota == j, spl, exv)
            ex_v[pl.ds(g * L, L)] = exv
            return carry2
        lax.fori_loop(0, GPC, group, 0)
        pltpu.async_copy(ex_v, ex_out.at[pl.ds(tile_base + ch * C, C)], sem_e)
        pltpu.async_copy(exrow_v, den_sh.at[idxq], sem_w, add=True)

    def sdrain(ch, idxq, ex_v, exrow_v, sem_w, sem_e):
        pltpu.make_async_copy(
            ex_v, ex_out.at[pl.ds(tile_base + ch * C, C)], sem_e).wait()
        pltpu.make_async_copy(
            exrow_v, den_sh.at[idxq], sem_w).wait()

    half = cpt // 2
    gathers(0, idxd0, rs0, rd0, sem_s0, sem_d0)

    def pair(t, carry):
        ch0 = 2 * t
        gwait(ch0, idxd0, rs0, rd0, sem_s0, sem_d0)
        gathers(ch0 + 1, idxd1, rs1, rd1, sem_s1, sem_d1)

        @pl.when(t > 0)
        def _():
            sdrain(ch0 - 2, idxq0, exv0, exrow0, sem_w0, sem_e0)
        compute(ch0, rs0, rd0, idxd0, idxq0, exv0, exrow0, sem_w0, sem_e0)
        gwait(ch0 + 1, idxd1, rs1, rd1, sem_s1, sem_d1)

        @pl.when(t + 1 < half)
        def _():
            gathers(ch0 + 2, idxd0, rs0, rd0, sem_s0, sem_d0)

        @pl.when(t > 0)
        def _():
            sdrain(ch0 - 1, idxq1, exv1, exrow1, sem_w1, sem_e1)
        compute(ch0 + 1, rs1, rd1, idxd1, idxq1, exv1, exrow1, sem_w1, sem_e1)
        return carry

    lax.fori_loop(0, half, pair, 0)
    sdrain(cpt - 2, idxq0, exv0, exrow0, sem_w0, sem_e0)
    sdrain(cpt - 1, idxq1, exv1, exrow1, sem_w1, sem_e1)

    plsc.subcore_barrier()
    pltpu.sync_copy(den_sh.at[pl.ds(s * per_tile, per_tile)],
                    den_out.at[c, pl.ds(s * per_tile, per_tile)])


def _phase2_body(cpt, d_dim, xpad, srcf, dstf, ex_hbm, agg_out,
                 src_all, idxd0, idxd1, r0, r1, ex_v,
                 agg_sh, sem_r0, sem_r1):
    c = lax.axis_index("c")
    s = lax.axis_index("s")
    w = s * NC + c
    npad = agg_sh.shape[0]
    per_tile = npad // NS

    def zfill(i, carry):
        for k in range(d_dim // L):
            r0[i, pl.ds(k * L, L)] = jnp.zeros((L,), jnp.float32)
        return carry
    lax.fori_loop(0, C, zfill, 0)

    for off in range(0, per_tile, C):
        wdt = min(C, per_tile - off)
        pltpu.sync_copy(r0.at[pl.ds(0, wdt)],
                        agg_sh.at[pl.ds(s * per_tile + off, wdt)])
    plsc.subcore_barrier()

    tile_base = w * (cpt * C)
    pltpu.sync_copy(srcf.at[pl.ds(tile_base, cpt * C)], src_all)

    def gather(ch, rr, sem_r):
        pltpu.async_copy(xpad.at[src_all.at[pl.ds(ch * C, C)]], rr, sem_r)

    def gwait(ch, rr, sem_r):
        pltpu.make_async_copy(xpad.at[src_all.at[pl.ds(ch * C, C)]],
                              rr, sem_r).wait()

    def compute(ch, rows_v, idxd):
        pltpu.sync_copy(dstf.at[pl.ds(tile_base + ch * C, C)], idxd)
        pltpu.sync_copy(ex_hbm.at[pl.ds(tile_base + ch * C, C)], ex_v)

        def group(g, carry2):
            exg = ex_v[pl.ds(g * L, L)]
            for j in range(L):
                e_ref = g * L + j
                spl = _rotg(exg, jnp.full((L,), j, jnp.int32))
                for k in range(d_dim // L):
                    v = rows_v[e_ref, pl.ds(k * L, L)]
                    rows_v[e_ref, pl.ds(k * L, L)] = v * spl
            return carry2
        lax.fori_loop(0, GPC, group, 0)
        pltpu.sync_copy(rows_v, agg_sh.at[idxd], add=True)

    half = cpt // 2
    gather(0, r0, sem_r0)

    def pair(t, carry):
        ch0 = 2 * t
        gwait(ch0, r0, sem_r0)
        gather(ch0 + 1, r1, sem_r1)
        compute(ch0, r0, idxd0)
        gwait(ch0 + 1, r1, sem_r1)

        @pl.when(t + 1 < half)
        def _():
            gather(ch0 + 2, r0, sem_r0)
        compute(ch0 + 1, r1, idxd1)
        return carry

    lax.fori_loop(0, half, pair, 0)

    plsc.subcore_barrier()
    pltpu.sync_copy(agg_sh.at[pl.ds(s * per_tile, per_tile)],
                    agg_out.at[c, pl.ds(s * per_tile, per_tile)])


def kernel(x, edge_index, beta):
    n, d = x.shape
    e = edge_index.shape[1]
    et = e + n
    cpt = -(-et // (NW * C))
    cpt = cpt + (cpt & 1)
    etpad = NW * C * cpt
    pad_e = etpad - et
    npad = (NS * 8) * (-(-(n + 1) // (NS * 8)))

    src = edge_index[0].astype(jnp.int32)
    dst = edge_index[1].astype(jnp.int32)
    loops = jnp.arange(n, dtype=jnp.int32)
    srcf = jnp.concatenate([src, loops, jnp.zeros((pad_e,), jnp.int32)])
    dstf = jnp.concatenate([dst, loops, jnp.full((pad_e,), n, jnp.int32)])
    xpad = jnp.pad(x, ((0, npad - n), (0, 0)))
    beta16 = jnp.broadcast_to(beta.astype(jnp.float32), (L,))

    rows_blk = 128
    xn = pl.pallas_call(
        _norm_body,
        grid=(npad // rows_blk,),
        in_specs=[pl.BlockSpec((rows_blk, d), lambda i: (i, 0))],
        out_specs=pl.BlockSpec((rows_blk, d), lambda i: (i, 0)),
        out_shape=jax.ShapeDtypeStruct((npad, d), jnp.float32),
    )(xpad)

    mesh = plsc.VectorSubcoreMesh(
        core_axis_name="c", subcore_axis_name="s",
        num_cores=NC, num_subcores=NS)

    nq = (NS * 8) * (-(-(npad // 8) // (NS * 8)))

    phase1 = pl.kernel(
        functools.partial(_phase1_body, cpt, d),
        out_type=(jax.ShapeDtypeStruct((NC, nq, d), jnp.float32),
                  jax.ShapeDtypeStruct((etpad,), jnp.float32)),
        mesh=mesh,
        scratch_types=[
            pltpu.VMEM((cpt * C,), jnp.int32),
            pltpu.VMEM((C,), jnp.int32),
            pltpu.VMEM((C,), jnp.int32),
            pltpu.VMEM((C,), jnp.int32),
            pltpu.VMEM((C,), jnp.int32),
            pltpu.VMEM((C, d), jnp.float32),
            pltpu.VMEM((C, d), jnp.float32),
            pltpu.VMEM((C, d), jnp.float32),
            pltpu.VMEM((C, d), jnp.float32),
            pltpu.VMEM((C,), jnp.float32),
            pltpu.VMEM((C,), jnp.float32),
            pltpu.VMEM((C, d), jnp.float32),
            pltpu.VMEM((C, d), jnp.float32),
            pltpu.VMEM((L,), jnp.float32),
            pltpu.VMEM_SHARED((nq, d), jnp.float32),
            pltpu.SemaphoreType.DMA,
            pltpu.SemaphoreType.DMA,
            pltpu.SemaphoreType.DMA,
            pltpu.SemaphoreType.DMA,
            pltpu.SemaphoreType.DMA,
            pltpu.SemaphoreType.DMA,
            pltpu.SemaphoreType.DMA,
            pltpu.SemaphoreType.DMA,
        ],
    )
    denp, ex = phase1(xn, srcf, dstf, beta16)
    dencol = jnp.broadcast_to(
        denp.reshape(NC, nq * d)[:, ::L][:, :n, None], (NC, n, d))

    phase2 = pl.kernel(
        functools.partial(_phase2_body, cpt, d),
        out_type=jax.ShapeDtypeStruct((NC, npad, d), jnp.float32),
        mesh=mesh,
        scratch_types=[
            pltpu.VMEM((cpt * C,), jnp.int32),
            pltpu.VMEM((C,), jnp.int32),
            pltpu.VMEM((C,), jnp.int32),
            pltpu.VMEM((C, d), jnp.float32),
            pltpu.VMEM((C, d), jnp.float32),
            pltpu.VMEM((C,), jnp.float32),
            pltpu.VMEM_SHARED((npad, d), jnp.float32),
            pltpu.SemaphoreType.DMA,
            pltpu.SemaphoreType.DMA,
        ],
    )
    aggp = phase2(xpad, srcf, dstf, ex)

    fb = 80
    out = pl.pallas_call(
        _final_body,
        grid=(n // fb,),
        in_specs=[pl.BlockSpec((fb, d), lambda i: (i, 0))] * 5,
        out_specs=pl.BlockSpec((fb, d), lambda i: (i, 0)),
        out_shape=jax.ShapeDtypeStruct((n, d), jnp.float32),
    )(x, aggp[0, :n], aggp[1, :n], dencol[0], dencol[1])
    return out


# speedup vs baseline: 7.4116x; 1.0225x over previous
import functools

import jax
import jax.numpy as jnp
from jax import lax
from jax.experimental import pallas as pl
from jax.experimental.pallas import tpu as pltpu
from jax.experimental.pallas import tpu_sc as plsc

L = 16
NC = 2
NS = 16
NW = NC * NS
C = 128
GPC = C // L


def _rotg(v, idx):
    return lax.gather(
        v, idx[:, None],
        dimension_numbers=lax.GatherDimensionNumbers(
            offset_dims=(), collapsed_slice_dims=(0,), start_index_map=(0,)),
        slice_sizes=(1,),
        mode=lax.GatherScatterMode.PROMISE_IN_BOUNDS)


def _norm_body(x_ref, o_ref):
    x = x_ref[...]
    ss = jnp.sum(x * x, axis=1, keepdims=True)
    nrm = jnp.maximum(jnp.sqrt(ss), 1e-12)
    o_ref[...] = x / nrm


def _final_body(x_ref, a0_ref, a1_ref, d0_ref, d1_ref, o_ref):
    num = a0_ref[...] + a1_ref[...]
    den = jnp.maximum(d0_ref[...] + d1_ref[...], 1e-16)
    o_ref[...] = x_ref[...] + jnp.maximum(num / den, 0.0)


def _phase1_body(cpt, d_dim, xn, srcf, dstf, beta16_hbm, den_out, ex_out,
                 src_all, idxd0, idxd1, idxq0, idxq1, rs0, rd0, rs1, rd1,
                 exv0, exv1, exrow0, exrow1, beta_v,
                 den_sh, sem_s0, sem_d0, sem_s1, sem_d1, sem_w0, sem_w1,
                 sem_e0, sem_e1):
    c = lax.axis_index("c")
    s = lax.axis_index("s")
    w = s * NC + c
    nq = den_sh.shape[0]
    per_tile = nq // NS
    iota = lax.broadcasted_iota(jnp.int32, (L,), 0)

    def zfill(i, carry):
        for k in range(d_dim // L):
            exrow0[i, pl.ds(k * L, L)] = jnp.zeros((L,), jnp.float32)
        return carry
    lax.fori_loop(0, C, zfill, 0)

    for off in range(0, per_tile, C):
        wdt = min(C, per_tile - off)
        pltpu.sync_copy(exrow0.at[pl.ds(0, wdt)],
                        den_sh.at[pl.ds(s * per_tile + off, wdt)])
    plsc.subcore_barrier()

    pltpu.sync_copy(beta16_hbm, beta_v)
    beta16 = beta_v[...]
    tile_base = w * (cpt * C)
    pltpu.sync_copy(srcf.at[pl.ds(tile_base, cpt * C)], src_all)

    def gathers(ch, idxd, rs, rd, sem_s, sem_d):
        pltpu.sync_copy(dstf.at[pl.ds(tile_base + ch * C, C)], idxd)
        pltpu.async_copy(xn.at[src_all.at[pl.ds(ch * C, C)]], rs, sem_s)
        pltpu.async_copy(xn.at[idxd], rd, sem_d)

    def gwait(ch, idxd, rs, rd, sem_s, sem_d):
        pltpu.make_async_copy(xn.at[src_all.at[pl.ds(ch * C, C)]],
                              rs, sem_s).wait()
        pltpu.make_async_copy(xn.at[idxd], rd, sem_d).wait()

    def compute(ch, rows_s, rows_d, idxd, idxq, ex_v, exrow_v, sem_w, sem_e):
        def group(g, carry2):
            dst16 = idxd[pl.ds(g * L, L)]
            idxq[pl.ds(g * L, L)] = lax.shift_right_logical(dst16, 3)
            exv = jnp.zeros((L,), jnp.float32)
            for j in range(L):
                e_ref = g * L + j
                acc = rows_s[e_ref, pl.ds(0, L)] * rows_d[e_ref, pl.ds(0, L)]
                for k in range(1, d_dim // L):
                    acc = acc + (rows_s[e_ref, pl.ds(k * L, L)] *
                                 rows_d[e_ref, pl.ds(k * L, L)])
                for r in (8, 4, 2, 1):
                    acc = acc + _rotg(acc, (iota + r) & (L - 1))
                spl = jnp.exp(acc * beta16)
                blk = _rotg(dst16, jnp.full((L,), j, jnp.int32)) & 7
                blkf = blk.astype(jnp.float32)
                for k in range(d_dim // L):
                    mk = jnp.maximum(
                        1.0 - jnp.abs(blkf - jnp.float32(k)), 0.0)
                    exrow_v[e_ref, pl.ds(k * L, L)] = spl * mk
                exv = jnp.where(iota == j, spl, exv)
            ex_v[pl.ds(g * L, L)] = exv
            return carry2
        lax.fori_loop(0, GPC, group, 0)
        pltpu.async_copy(ex_v, ex_out.at[pl.ds(tile_base + ch * C, C)], sem_e)
        pltpu.async_copy(exrow_v, den_sh.at[idxq], sem_w, add=True)

    def sdrain(ch, idxq, ex_v, exrow_v, sem_w, sem_e):
        pltpu.make_async_copy(
            ex_v, ex_out.at[pl.ds(tile_base + ch * C, C)], sem_e).wait()
        pltpu.make_async_copy(
            exrow_v, den_sh.at[idxq], sem_w).wait()

    half = cpt // 2
    gathers(0, idxd0, rs0, rd0, sem_s0, sem_d0)

    def pair(t, carry):
        ch0 = 2 * t
        gwait(ch0, idxd0, rs0, rd0, sem_s0, sem_d0)
        gathers(ch0 + 1, idxd1, rs1, rd1, sem_s1, sem_d1)

        @pl.when(t > 0)
        def _():
            sdrain(ch0 - 2, idxq0, exv0, exrow0, sem_w0, sem_e0)
        compute(ch0, rs0, rd0, idxd0, idxq0, exv0, exrow0, sem_w0, sem_e0)
        gwait(ch0 + 1, idxd1, rs1, rd1, sem_s1, sem_d1)

        @pl.when(t + 1 < half)
        def _():
            gathers(ch0 + 2, idxd0, rs0, rd0, sem_s0, sem_d0)

        @pl.when(t > 0)
        def _():
            sdrain(ch0 - 1, idxq1, exv1, exrow1, sem_w1, sem_e1)
        compute(ch0 + 1, rs1, rd1, idxd1, idxq1, exv1, exrow1, sem_w1, sem_e1)
        return carry

    lax.fori_loop(0, half, pair, 0)
    sdrain(cpt - 2, idxq0, exv0, exrow0, sem_w0, sem_e0)
    sdrain(cpt - 1, idxq1, exv1, exrow1, sem_w1, sem_e1)

    plsc.subcore_barrier()
    pltpu.sync_copy(den_sh.at[pl.ds(s * per_tile, per_tile)],
                    den_out.at[c, pl.ds(s * per_tile, per_tile)])


def _phase2_body(cpt, d_dim, xpad, srcf, dstf, ex_hbm, agg_out,
                 src_all, idxd0, idxd1, r0, r1, exv0, exv1,
                 agg_sh, sem_r0, sem_r1, sem_i0, sem_i1, sem_x0, sem_x1):
    c = lax.axis_index("c")
    s = lax.axis_index("s")
    w = s * NC + c
    npad = agg_sh.shape[0]
    per_tile = npad // NS

    def zfill(i, carry):
        for k in range(d_dim // L):
            r0[i, pl.ds(k * L, L)] = jnp.zeros((L,), jnp.float32)
        return carry
    lax.fori_loop(0, C, zfill, 0)

    for off in range(0, per_tile, C):
        wdt = min(C, per_tile - off)
        pltpu.sync_copy(r0.at[pl.ds(0, wdt)],
                        agg_sh.at[pl.ds(s * per_tile + off, wdt)])
    plsc.subcore_barrier()

    tile_base = w * (cpt * C)
    pltpu.sync_copy(srcf.at[pl.ds(tile_base, cpt * C)], src_all)

    def gather(ch, rr, idxd, ex_v, sem_r, sem_i, sem_x):
        base = tile_base + ch * C
        pltpu.async_copy(xpad.at[src_all.at[pl.ds(ch * C, C)]], rr, sem_r)
        pltpu.async_copy(dstf.at[pl.ds(base, C)], idxd, sem_i)
        pltpu.async_copy(ex_hbm.at[pl.ds(base, C)], ex_v, sem_x)

    def gwait(ch, rr, idxd, ex_v, sem_r, sem_i, sem_x):
        base = tile_base + ch * C
        pltpu.make_async_copy(xpad.at[src_all.at[pl.ds(ch * C, C)]],
                              rr, sem_r).wait()
        pltpu.make_async_copy(dstf.at[pl.ds(base, C)], idxd, sem_i).wait()
        pltpu.make_async_copy(ex_hbm.at[pl.ds(base, C)], ex_v, sem_x).wait()

    def compute(ch, rows_v, idxd, ex_v):
        def group(g, carry2):
            exg = ex_v[pl.ds(g * L, L)]
            for j in range(L):
                e_ref = g * L + j
                spl = _rotg(exg, jnp.full((L,), j, jnp.int32))
                for k in range(d_dim // L):
                    v = rows_v[e_ref, pl.ds(k * L, L)]
                    rows_v[e_ref, pl.ds(k * L, L)] = v * spl
            return carry2
        lax.fori_loop(0, GPC, group, 0)
        pltpu.sync_copy(rows_v, agg_sh.at[idxd], add=True)

    half = cpt // 2
    gather(0, r0, idxd0, exv0, sem_r0, sem_i0, sem_x0)

    def pair(t, carry):
        ch0 = 2 * t
        gwait(ch0, r0, idxd0, exv0, sem_r0, sem_i0, sem_x0)
        gather(ch0 + 1, r1, idxd1, exv1, sem_r1, sem_i1, sem_x1)
        compute(ch0, r0, idxd0, exv0)
        gwait(ch0 + 1, r1, idxd1, exv1, sem_r1, sem_i1, sem_x1)

        @pl.when(t + 1 < half)
        def _():
            gather(ch0 + 2, r0, idxd0, exv0, sem_r0, sem_i0, sem_x0)
        compute(ch0 + 1, r1, idxd1, exv1)
        return carry

    lax.fori_loop(0, half, pair, 0)

    plsc.subcore_barrier()
    pltpu.sync_copy(agg_sh.at[pl.ds(s * per_tile, per_tile)],
                    agg_out.at[c, pl.ds(s * per_tile, per_tile)])


def kernel(x, edge_index, beta):
    n, d = x.shape
    e = edge_index.shape[1]
    et = e + n
    cpt = -(-et // (NW * C))
    cpt = cpt + (cpt & 1)
    etpad = NW * C * cpt
    pad_e = etpad - et
    npad = (NS * 8) * (-(-(n + 1) // (NS * 8)))

    src = edge_index[0].astype(jnp.int32)
    dst = edge_index[1].astype(jnp.int32)
    loops = jnp.arange(n, dtype=jnp.int32)
    srcf = jnp.concatenate([src, loops, jnp.zeros((pad_e,), jnp.int32)])
    dstf = jnp.concatenate([dst, loops, jnp.full((pad_e,), n, jnp.int32)])
    xpad = jnp.pad(x, ((0, npad - n), (0, 0)))
    beta16 = jnp.broadcast_to(beta.astype(jnp.float32), (L,))

    rows_blk = 128
    xn = pl.pallas_call(
        _norm_body,
        grid=(npad // rows_blk,),
        in_specs=[pl.BlockSpec((rows_blk, d), lambda i: (i, 0))],
        out_specs=pl.BlockSpec((rows_blk, d), lambda i: (i, 0)),
        out_shape=jax.ShapeDtypeStruct((npad, d), jnp.float32),
    )(xpad)

    mesh = plsc.VectorSubcoreMesh(
        core_axis_name="c", subcore_axis_name="s",
        num_cores=NC, num_subcores=NS)

    nq = (NS * 8) * (-(-(npad // 8) // (NS * 8)))

    phase1 = pl.kernel(
        functools.partial(_phase1_body, cpt, d),
        out_type=(jax.ShapeDtypeStruct((NC, nq, d), jnp.float32),
                  jax.ShapeDtypeStruct((etpad,), jnp.float32)),
        mesh=mesh,
        scratch_types=[
            pltpu.VMEM((cpt * C,), jnp.int32),
            pltpu.VMEM((C,), jnp.int32),
            pltpu.VMEM((C,), jnp.int32),
            pltpu.VMEM((C,), jnp.int32),
            pltpu.VMEM((C,), jnp.int32),
            pltpu.VMEM((C, d), jnp.float32),
            pltpu.VMEM((C, d), jnp.float32),
            pltpu.VMEM((C, d), jnp.float32),
            pltpu.VMEM((C, d), jnp.float32),
            pltpu.VMEM((C,), jnp.float32),
            pltpu.VMEM((C,), jnp.float32),
            pltpu.VMEM((C, d), jnp.float32),
            pltpu.VMEM((C, d), jnp.float32),
            pltpu.VMEM((L,), jnp.float32),
            pltpu.VMEM_SHARED((nq, d), jnp.float32),
            pltpu.SemaphoreType.DMA,
            pltpu.SemaphoreType.DMA,
            pltpu.SemaphoreType.DMA,
            pltpu.SemaphoreType.DMA,
            pltpu.SemaphoreType.DMA,
            pltpu.SemaphoreType.DMA,
            pltpu.SemaphoreType.DMA,
            pltpu.SemaphoreType.DMA,
        ],
    )
    denp, ex = phase1(xn, srcf, dstf, beta16)
    dencol = jnp.broadcast_to(
        denp.reshape(NC, nq * d)[:, ::L][:, :n, None], (NC, n, d))

    phase2 = pl.kernel(
        functools.partial(_phase2_body, cpt, d),
        out_type=jax.ShapeDtypeStruct((NC, npad, d), jnp.float32),
        mesh=mesh,
        scratch_types=[
            pltpu.VMEM((cpt * C,), jnp.int32),
            pltpu.VMEM((C,), jnp.int32),
            pltpu.VMEM((C,), jnp.int32),
            pltpu.VMEM((C, d), jnp.float32),
            pltpu.VMEM((C, d), jnp.float32),
            pltpu.VMEM((C,), jnp.float32),
            pltpu.VMEM((C,), jnp.float32),
            pltpu.VMEM_SHARED((npad, d), jnp.float32),
            pltpu.SemaphoreType.DMA,
            pltpu.SemaphoreType.DMA,
            pltpu.SemaphoreType.DMA,
            pltpu.SemaphoreType.DMA,
            pltpu.SemaphoreType.DMA,
            pltpu.SemaphoreType.DMA,
        ],
    )
    aggp = phase2(xpad, srcf, dstf, ex)

    fb = 80
    out = pl.pallas_call(
        _final_body,
        grid=(n // fb,),
        in_specs=[pl.BlockSpec((fb, d), lambda i: (i, 0))] * 5,
        out_specs=pl.BlockSpec((fb, d), lambda i: (i, 0)),
        out_shape=jax.ShapeDtypeStruct((n, d), jnp.float32),
    )(x, aggp[0, :n], aggp[1, :n], dencol[0], dencol[1])
    return out
